# Initial kernel scaffold; baseline (speedup 1.0000x reference)
import jax
import jax.numpy as jnp
from jax.experimental import pallas as pl


def kernel(x, edge_index, turn_ids, W_gat, a_src, a_dst, Wc, Ws, W_ih, W_hh, b_ih, b_hh, fc_W, fc_b, score_W, score_b):
    raise NotImplementedError("write your pallas kernel here")


# speedup vs baseline: 143.8475x; 143.8475x over previous
import functools

import jax
import jax.numpy as jnp
from jax import lax
from jax.experimental import pallas as pl
from jax.experimental.pallas import tpu as pltpu
from jax.experimental.pallas import tpu_sc as plsc

N = 10000
E = 320000
NFEAT = 128
NHID = 256
NHEAD = 8
DH = 32
T = 10
ALPHA = 0.2
TN = T * N

BN_A = 400
NB_A = N // BN_A


def _stage_a_body(x_ref, wg_ref, asrc_ref, adst_ref, wh2_ref, es_ref, ed_ref):
    h = pl.program_id(2)
    wh = jnp.dot(x_ref[...], wg_ref[0], preferred_element_type=jnp.float32,
                 precision=lax.Precision.DEFAULT)
    wh2_ref[0] = wh
    es = jnp.dot(wh, asrc_ref[0], preferred_element_type=jnp.float32,
                  precision=lax.Precision.HIGHEST)
    ed = jnp.dot(wh, adst_ref[0], preferred_element_type=jnp.float32,
                  precision=lax.Precision.HIGHEST)

    @pl.when(h == 0)
    def _():
        es_ref[...] = es
        ed_ref[...] = ed

    @pl.when(h == 1)
    def _():
        es_ref[...] += es
        ed_ref[...] += ed


def _stage_a(x, W_gat, Asrc, Adst):
    return pl.pallas_call(
        _stage_a_body,
        grid=(NB_A, T, 2),
        in_specs=[
            pl.BlockSpec((BN_A, NFEAT), lambda i, t, h: (i, 0)),
            pl.BlockSpec((1, NFEAT, 128), lambda i, t, h: (t, 0, h)),
            pl.BlockSpec((1, 128, 128), lambda i, t, h: (t, h, 0)),
            pl.BlockSpec((1, 128, 128), lambda i, t, h: (t, h, 0)),
        ],
        out_specs=[
            pl.BlockSpec((1, BN_A, 128), lambda i, t, h: (h, t * NB_A + i, 0)),
            pl.BlockSpec((BN_A, 128), lambda i, t, h: (t * NB_A + i, 0)),
            pl.BlockSpec((BN_A, 128), lambda i, t, h: (t * NB_A + i, 0)),
        ],
        out_shape=[
            jax.ShapeDtypeStruct((2, TN, 128), jnp.float32),
            jax.ShapeDtypeStruct((TN, 128), jnp.float32),
            jax.ShapeDtypeStruct((TN, 128), jnp.float32),
        ],
    )(x, W_gat, Asrc, Adst)


KE = 64
NTILES = E // KE
NSUB = 16
TEXTRA = NTILES - (NTILES // NSUB) * NSUB
NT2 = NTILES // 2
T2EXTRA = NT2 - (NT2 // NSUB) * NSUB
ZB = N // 16
ZBEXTRA = ZB - (ZB // NSUB) * NSUB


def _edge_body(meta_h, src_h, dst_h, tid_h, es_h, ed_h, wh_h,
               aggu_h, dnm_h,
               src_v, dst_v, t_v, fidx_v, didx_v, widx_v, zidx_v,
               es_v, ed_v, wh_v, z1, acc_sp):
    c = lax.axis_index("c")
    s = lax.axis_index("s")
    lane16 = lax.iota(jnp.int32, 16)

    def z1loop(i, _):
        z1[i // 8, pl.ds(16 * (i % 8), 16)] = jnp.zeros((16,), jnp.float32)
        return 0
    lax.fori_loop(0, 16 * 8, z1loop, 0)

    n_zb = jnp.where(s < ZBEXTRA, ZB // NSUB + 1, ZB // NSUB)

    def zero_acc():
        def zbatch(k, _):
            zidx_v[pl.ds(0, 16)] = (s + NSUB * k) * 16 + lane16
            pltpu.sync_copy(z1.at[pl.ds(0, 16)], acc_sp.at[zidx_v])
            return 0
        lax.fori_loop(0, n_zb, zbatch, 0)

    def copy_out(dst_h):
        def obatch(k, _):
            base = (s + NSUB * k) * 16
            zidx_v[pl.ds(0, 16)] = base + lane16
            pltpu.sync_copy(acc_sp.at[zidx_v], wh_v.at[pl.ds(0, 16)])
            pltpu.sync_copy(wh_v.at[pl.ds(0, 16)], dst_h.at[c, pl.ds(base, 16)])
            return 0
        lax.fori_loop(0, n_zb, obatch, 0)

    def load_indices(off):
        pltpu.sync_copy(src_h.at[pl.ds(off, KE)], src_v)
        pltpu.sync_copy(dst_h.at[pl.ds(off, KE)], dst_v)
        pltpu.sync_copy(tid_h.at[dst_v], t_v)
        for i in range(KE // 16):
            sl = src_v[pl.ds(16 * i, 16)]
            dl = dst_v[pl.ds(16 * i, 16)]
            tn = t_v[pl.ds(16 * i, 16)] * N
            fidx_v[pl.ds(16 * i, 16)] = tn + sl
            didx_v[pl.ds(16 * i, 16)] = tn + dl
            widx_v[pl.ds(16 * i, 16)] = tn + sl + c * TN

    zero_acc()
    plsc.subcore_barrier()

    def tile_body(jj, _):
        off = (s + NSUB * jj) * KE
        load_indices(off)
        pltpu.sync_copy(es_h.at[fidx_v], es_v)
        pltpu.sync_copy(ed_h.at[didx_v], ed_v)
        pltpu.sync_copy(wh_h.at[widx_v], wh_v)

        def edge_body(e, _):
            v = es_v[e, pl.ds(0, 16)] + ed_v[e, pl.ds(0, 16)]
            ex = jnp.exp(jnp.maximum(v, ALPHA * v))
            for h in range(4):
                sval = jnp.where(c == 0, ex[h], ex[4 + h])
                sc = jnp.broadcast_to(sval, (16,))
                for q in range(2):
                    jcol = 16 * (2 * h + q)
                    wh_v[e, pl.ds(jcol, 16)] = wh_v[e, pl.ds(jcol, 16)] * sc
            return 0
        lax.fori_loop(0, KE, edge_body, 0)
        pltpu.sync_copy(wh_v, acc_sp.at[dst_v], add=True)
        return 0

    n_tiles = jnp.where(s < TEXTRA, NTILES // NSUB + 1, NTILES // NSUB)
    lax.fori_loop(0, n_tiles, tile_body, 0)
    plsc.subcore_barrier()
    copy_out(aggu_h)
    plsc.subcore_barrier()

    def zwloop(i, _):
        wh_v[i // 8, pl.ds(16 * (i % 8), 16)] = jnp.zeros((16,), jnp.float32)
        return 0
    lax.fori_loop(0, KE * 8, zwloop, 0)
    zero_acc()
    plsc.subcore_barrier()

    def tile2_body(jj, _):
        off = (c * NT2 + s + NSUB * jj) * KE
        load_indices(off)
        pltpu.sync_copy(es_h.at[fidx_v], es_v)
        pltpu.sync_copy(ed_h.at[didx_v], ed_v)

        def edge2_body(e, _):
            v = es_v[e, pl.ds(0, 16)] + ed_v[e, pl.ds(0, 16)]
            ex = jnp.exp(jnp.maximum(v, ALPHA * v))
            wh_v[e, pl.ds(0, 16)] = ex
            return 0
        lax.fori_loop(0, KE, edge2_body, 0)
        pltpu.sync_copy(wh_v, acc_sp.at[dst_v], add=True)
        return 0

    n_t2 = jnp.where(s < T2EXTRA, NT2 // NSUB + 1, NT2 // NSUB)
    lax.fori_loop(0, n_t2, tile2_body, 0)
    plsc.subcore_barrier()
    copy_out(dnm_h)


def _edge_pass(meta, src, dst, tid, es_all, ed_all, wh2):
    mesh = plsc.VectorSubcoreMesh(core_axis_name="c", subcore_axis_name="s",
                                  num_cores=2, num_subcores=NSUB)
    f = pl.kernel(
        _edge_body,
        out_type=(jax.ShapeDtypeStruct((2, N, 128), jnp.float32),
                  jax.ShapeDtypeStruct((2, N, 128), jnp.float32)),
        mesh=mesh,
        scratch_types=[
            pltpu.VMEM((KE,), jnp.int32),
            pltpu.VMEM((KE,), jnp.int32),
            pltpu.VMEM((KE,), jnp.int32),
            pltpu.VMEM((KE,), jnp.int32),
            pltpu.VMEM((KE,), jnp.int32),
            pltpu.VMEM((KE,), jnp.int32),
            pltpu.VMEM((16,), jnp.int32),
            pltpu.VMEM((KE, 128), jnp.float32),
            pltpu.VMEM((KE, 128), jnp.float32),
            pltpu.VMEM((KE, 128), jnp.float32),
            pltpu.VMEM((16, 128), jnp.float32),
            pltpu.VMEM_SHARED((N, 128), jnp.float32),
        ],
    )
    return f(meta, src, dst, tid, es_all, ed_all, wh2)


BN_C = 1000


def _stage_c_body(agg_ref, dn_ref, hp_ref):
    h = pl.program_id(1)
    den128 = dn_ref[0] + dn_ref[1]
    hexp = (jax.lax.broadcasted_iota(jnp.int32, (128, 128), 0)
            == 4 * h + jax.lax.broadcasted_iota(jnp.int32, (128, 128), 1) // DH)
    den = jnp.dot(den128, hexp.astype(jnp.float32),
                  preferred_element_type=jnp.float32,
                  precision=lax.Precision.HIGHEST)
    a = agg_ref[0] / (den + 1e-9)
    hp_ref[...] = jnp.where(a > 0, a, jnp.exp(jnp.minimum(a, 0.0)) - 1.0)


def _stage_c(aggu, dnm):
    return pl.pallas_call(
        _stage_c_body,
        grid=(N // BN_C, 2),
        in_specs=[pl.BlockSpec((1, BN_C, 128), lambda i, h: (h, i, 0)),
                  pl.BlockSpec((2, BN_C, 128), lambda i, h: (0, i, 0))],
        out_specs=pl.BlockSpec((BN_C, 128), lambda i, h: (i, h)),
        out_shape=jax.ShapeDtypeStruct((N, NHID), jnp.float32),
    )(aggu, dnm)


BN_D = 400
NB_D = N // BN_D


def _stage_d_body(meta_ref, hp_ref, whh_ref, gh_ref):
    i = pl.program_id(0)
    t = pl.program_id(1)

    @pl.when(t == 0)
    def _():
        gh_ref[...] = jnp.zeros_like(gh_ref)

    lo = meta_ref[t + 1]
    hi = meta_ref[t + 2]

    @pl.when((hi > i * BN_D) & (lo < (i + 1) * BN_D))
    def _():
        r = jax.lax.broadcasted_iota(jnp.int32, (BN_D, NHID), 0) + i * BN_D
        m = (r >= lo) & (r < hi)
        hpm = jnp.where(m, hp_ref[...], 0.0)
        gh_ref[...] += jnp.dot(hpm, whh_ref[t],
                               preferred_element_type=jnp.float32,
                               precision=lax.Precision.DEFAULT)


def _stage_d(meta, hp_gat, W_hh):
    return pl.pallas_call(
        _stage_d_body,
        grid=(NB_D, T - 1),
        in_specs=[
            pl.BlockSpec(memory_space=pltpu.SMEM),
            pl.BlockSpec((BN_D, NHID), lambda i, t: (i, 0)),
            pl.BlockSpec((T - 1, NHID, 3 * NHID), lambda i, t: (0, 0, 0)),
        ],
        out_specs=pl.BlockSpec((BN_D, 3 * NHID), lambda i, t: (i, 0)),
        out_shape=jax.ShapeDtypeStruct((N, 3 * NHID), jnp.float32),
    )(meta, hp_gat, W_hh)


BN_E = 1000
NB_E = N // BN_E


def _stage_e_body(meta_ref, hp0_ref, gh_ref, wc_ref, ws_ref, wih_ref,
                  bih_ref, bhh_ref, fcw_ref, fcb_ref, scw_ref, scb_ref,
                  out_ref, hp_v, pool_v, acc_v):
    s = pl.program_id(0)
    t = s + 1

    def pooled_sum(lo, hi, dstrow):
        acc_v[dstrow, :] = jnp.zeros((NHID,), jnp.float32)
        for b in range(NB_E):
            @pl.when((hi > b * BN_E) & (lo < (b + 1) * BN_E))
            def _():
                r = jax.lax.broadcasted_iota(
                    jnp.int32, (BN_E, NHID), 0) + b * BN_E
                m = (r >= lo) & (r < hi)
                blk = jnp.where(m, hp_v[pl.ds(b * BN_E, BN_E), :], 0.0)
                acc_v[dstrow, :] += jnp.sum(blk, axis=0)

    @pl.when(s == 0)
    def _():
        hp_v[...] = hp0_ref[...]
        lo0 = meta_ref[0]
        hi0 = meta_ref[1]
        pooled_sum(lo0, hi0, 0)
        cnt0 = (hi0 - lo0).astype(jnp.float32)
        pool_v[0, :] = acc_v[0, :] * (1.0 / (cnt0 + 1e-9))

    lo = meta_ref[t]
    hi = meta_ref[t + 1]

    p1 = pool_v[pl.ds(s, 1), :]
    p2 = pool_v[pl.ds(jnp.maximum(s - 1, 0), 1), :]
    xin = jnp.tanh(jnp.dot(p1, wc_ref[0], preferred_element_type=jnp.float32,
                  precision=lax.Precision.DEFAULT))
    ws_mat = ws_ref[0]
    xin = xin + jnp.where(
        s > 0,
        jnp.tanh(jnp.dot(p2, ws_mat, preferred_element_type=jnp.float32,
                  precision=lax.Precision.DEFAULT)),
        0.0)
    gi = (jnp.dot(xin, wih_ref[0], preferred_element_type=jnp.float32,
                  precision=lax.Precision.DEFAULT)
          + bih_ref[pl.ds(s, 1), :])
    gz = gi[:, :NHID]
    gr = gi[:, NHID:2 * NHID]
    gn = gi[:, 2 * NHID:]
    bh = bhh_ref[pl.ds(s, 1), :]
    bz = bh[:, :NHID]
    br = bh[:, NHID:2 * NHID]
    bn = bh[:, 2 * NHID:]

    for b in range(NB_E):
        @pl.when((hi > b * BN_E) & (lo < (b + 1) * BN_E))
        def _():
            r = jax.lax.broadcasted_iota(jnp.int32, (BN_E, NHID), 0) + b * BN_E
            m = (r >= lo) & (r < hi)
            hpb = hp_v[pl.ds(b * BN_E, BN_E), :]
            ghb = gh_ref[pl.ds(b * BN_E, BN_E), :]
            z = jax.nn.sigmoid(gz + bz + ghb[:, :NHID])
            rg = jax.nn.sigmoid(gr + br + ghb[:, NHID:2 * NHID])
            nn = jnp.tanh(gn + bn + rg * ghb[:, 2 * NHID:])
            hf = (1.0 - z) * nn + z * hpb
            hp_v[pl.ds(b * BN_E, BN_E), :] = jnp.where(m, hf, hpb)

    pooled_sum(lo, hi, 1)
    cnt = (hi - lo).astype(jnp.float32)
    pool_v[pl.ds(t, 1), :] = (acc_v[pl.ds(1, 1), :]
                              * (1.0 / (cnt + 1e-9)))

    @pl.when(s == T - 2)
    def _():
        mt = meta_ref[11]
        ok1 = mt >= 2
        lo1 = jnp.where(ok1, meta_ref[jnp.maximum(mt - 2, 0)], 0)
        hi1 = jnp.where(ok1, meta_ref[jnp.maximum(mt - 1, 0)], 0)
        lo2 = meta_ref[jnp.maximum(mt - 1, 0)]
        hi2 = meta_ref[mt]
        pooled_sum(lo1, hi1, 2)
        pooled_sum(lo2, hi2, 3)
        h1 = acc_v[pl.ds(2, 1), :] * (1.0 / ((hi1 - lo1).astype(jnp.float32)
                                             + 1e-9))
        h2 = acc_v[pl.ds(3, 1), :] * (1.0 / ((hi2 - lo2).astype(jnp.float32)
                                             + 1e-9))
        def score(h):
            u = jnp.dot(h, fcw_ref[...], preferred_element_type=jnp.float32,
                  precision=lax.Precision.DEFAULT)
            u = jnp.maximum(u + fcb_ref[pl.ds(0, 1), :], 0.0)
            return (jnp.dot(u, scw_ref[...],
                            preferred_element_type=jnp.float32,
                  precision=lax.Precision.DEFAULT)
                    + scb_ref[0, 0])
        s1 = score(h1)
        s2 = score(h2)
        ci = jax.lax.broadcasted_iota(jnp.int32, (1, 128), 1)
        out_ref[...] = jnp.where(ci == 0, s1[0, 0],
                                 jnp.where(ci == 1, s2[0, 0], 0.0))


def _stage_e(meta, hp_gat, gh, Wc, Ws, W_ih, b_ih, b_hh, fc_W, fc_b,
             score_W, score_b):
    full = lambda shape: pl.BlockSpec(shape, lambda s: tuple(0 for _ in shape))
    return pl.pallas_call(
        _stage_e_body,
        grid=(T - 1,),
        in_specs=[
            pl.BlockSpec(memory_space=pltpu.SMEM),
            full((N, NHID)),
            full((N, 3 * NHID)),
            pl.BlockSpec((1, NHID, NHID), lambda s: (s, 0, 0)),
            pl.BlockSpec((1, NHID, NHID), lambda s: (jnp.maximum(s - 1, 0), 0, 0)),
            pl.BlockSpec((1, NHID, 3 * NHID), lambda s: (s, 0, 0)),
            full((T - 1, 3 * NHID)),
            full((T - 1, 3 * NHID)),
            full((NHID, 2 * NHID)),
            full((1, 2 * NHID)),
            full((2 * NHID, 1)),
            full((1, 1)),
        ],
        out_specs=full((1, 128)),
        out_shape=jax.ShapeDtypeStruct((1, 128), jnp.float32),
        scratch_shapes=[
            pltpu.VMEM((N, NHID), jnp.float32),
            pltpu.VMEM((16, NHID), jnp.float32),
            pltpu.VMEM((8, NHID), jnp.float32),
        ],
    )(meta, hp_gat, gh, Wc, Ws, W_ih, b_ih, b_hh, fc_W, fc_b,
      score_W, score_b)


def kernel(x, edge_index, turn_ids, W_gat, a_src, a_dst, Wc, Ws, W_ih, W_hh,
           b_ih, b_hh, fc_W, fc_b, score_W, score_b):
    src = edge_index[0].astype(jnp.int32)
    dst = edge_index[1].astype(jnp.int32)
    tid = turn_ids.astype(jnp.int32)

    onehot = (jnp.arange(NHID)[:, None] // DH == jnp.arange(16)[None, :])
    Asrc = a_src.reshape(T, NHID)[:, :, None] * onehot[None].astype(jnp.float32)
    Adst = a_dst.reshape(T, NHID)[:, :, None] * onehot[None].astype(jnp.float32)
    Asrc = jnp.pad(Asrc, ((0, 0), (0, 0), (0, 112)))
    Adst = jnp.pad(Adst, ((0, 0), (0, 0), (0, 112)))

    offs = jnp.searchsorted(tid, jnp.arange(T + 1, dtype=jnp.int32)
                            ).astype(jnp.int32)
    meta = jnp.concatenate([offs, (tid[-1] + 1)[None],
                            jnp.zeros(4, jnp.int32)])

    wh2, es_all, ed_all = _stage_a(x, W_gat, Asrc, Adst)
    aggu, dnm = _edge_pass(meta, src, dst, tid, es_all, ed_all,
                           wh2.reshape(2 * TN, 128))
    hp_gat = _stage_c(aggu, dnm)
    gh = _stage_d(meta, hp_gat, W_hh)
    out = _stage_e(meta, hp_gat, gh, Wc, Ws, W_ih, b_ih, b_hh, fc_W,
                   fc_b.reshape(1, 2 * NHID), score_W,
                   score_b.reshape(1, 1))
    return out[0, 0], out[0, 1]
